# Initial kernel scaffold; baseline (speedup 1.0000x reference)
import jax
import jax.numpy as jnp
from jax.experimental import pallas as pl


def kernel(CUSTOMER_CODE, ACTION_ID, WEIGHT_int, TIMES, customer_table, action_table, weight_table, time_table, time_mean, time_var, boundaries):
    raise NotImplementedError("write your pallas kernel here")


# speedup vs baseline: 2.3435x; 2.3435x over previous
import functools

import jax
import jax.numpy as jnp
from jax import lax
from jax.experimental import pallas as pl
from jax.experimental.pallas import tpu as pltpu
from jax.experimental.pallas import tpu_sc as plsc

B = 16384
D = 64
NBOUND = 1100
OUT_D = 3 * D + 1 + D
PAD_D = 264
TW = 72
L = 16
NC = 2
NS = 16
NW = NC * NS
ROWS_PER_W = B // NW
NB = 128
NCHUNK = ROWS_PER_W // NB

_mesh = plsc.VectorSubcoreMesh(core_axis_name="c", subcore_axis_name="s")


@functools.partial(
    pl.kernel,
    out_type=jax.ShapeDtypeStruct((B, PAD_D), jnp.float32),
    mesh=_mesh,
    compiler_params=pltpu.CompilerParams(use_tc_tiling_on_sc=False,
                                         needs_layout_passes=False),
    scratch_types=[
        pltpu.VMEM((NB,), jnp.int32),
        pltpu.VMEM((NB,), jnp.int32),
        pltpu.VMEM((NB,), jnp.int32),
        pltpu.VMEM((NB,), jnp.float32),
        pltpu.VMEM((NBOUND,), jnp.float32),
        pltpu.VMEM((4, L), jnp.float32),
        pltpu.VMEM((NB, D), jnp.float32),
        pltpu.VMEM((NB, D), jnp.float32),
        pltpu.VMEM((NB, D), jnp.float32),
        pltpu.VMEM((NB, TW), jnp.float32),
        pltpu.SemaphoreType.DMA,
    ],
)
def _sc_lookup(cidx_hbm, aidx_hbm, widx_hbm, times_hbm,
               ctab_hbm, atab_hbm, wtab_hbm, ttab_hbm,
               bounds_hbm, params_hbm, out_hbm,
               cidx_v, aidx_v, widx_v, times_v, bounds_v, params_v,
               crows_v, arows_v, wrows_v, trows_v, sem):
    wid = lax.axis_index("s") * NC + lax.axis_index("c")
    base_w = wid * ROWS_PER_W

    pltpu.sync_copy(bounds_hbm, bounds_v)
    pltpu.sync_copy(params_hbm, params_v)
    mean = params_v[0]
    inv = params_v[1]
    b0 = params_v[2]
    istep = params_v[3]

    def chunk_body(c, carry):
        r0 = base_w + c * NB
        pltpu.sync_copy(cidx_hbm.at[pl.ds(r0, NB)], cidx_v)
        pltpu.sync_copy(aidx_hbm.at[pl.ds(r0, NB)], aidx_v)
        pltpu.sync_copy(widx_hbm.at[pl.ds(r0, NB)], widx_v)
        pltpu.sync_copy(times_hbm.at[pl.ds(r0, NB)], times_v)

        g1 = pltpu.async_copy(ctab_hbm.at[cidx_v], crows_v, sem)
        g2 = pltpu.async_copy(atab_hbm.at[aidx_v], arows_v, sem)
        g3 = pltpu.async_copy(wtab_hbm.at[widx_v], wrows_v, sem)

        descs = []
        for i in range(NB // L):
            t = times_v[pl.ds(i * L, L)]
            g = ((t - b0) * istep + 0.5).astype(jnp.int32)
            m = jnp.clip(g - 2, 0, NBOUND - 4)
            cnt = jnp.zeros((L,), jnp.int32)
            for k in range(4):
                bk = plsc.load_gather(bounds_v, [m + k])
                cnt = cnt + jnp.where(bk < t, 1, 0)
            bins = m + cnt
            descs.append(pltpu.async_copy(
                ttab_hbm.at[bins], trows_v.at[pl.ds(i * L, L)], sem))

        g1.wait()
        g2.wait()
        g3.wait()
        for g in descs:
            g.wait()

        def norm(i, acc):
            t = times_v[pl.ds(i * L, L)]
            tn = (t - mean) * inv
            rows = i * L + lax.iota(jnp.int32, L)
            plsc.store_scatter(trows_v, [rows, jnp.zeros((L,), jnp.int32)], tn)
            return acc

        lax.fori_loop(0, NB // L, norm, 0)

        pltpu.sync_copy(crows_v, out_hbm.at[pl.ds(r0, NB), pl.ds(0, D)])
        pltpu.sync_copy(arows_v, out_hbm.at[pl.ds(r0, NB), pl.ds(D, D)])
        pltpu.sync_copy(wrows_v, out_hbm.at[pl.ds(r0, NB), pl.ds(2 * D, D)])
        pltpu.sync_copy(trows_v, out_hbm.at[pl.ds(r0, NB), pl.ds(3 * D, TW)])
        return carry

    lax.fori_loop(0, NCHUNK, chunk_body, 0)


def kernel(CUSTOMER_CODE, ACTION_ID, WEIGHT_int, TIMES, customer_table,
           action_table, weight_table, time_table, time_mean, time_var,
           boundaries):
    f32 = jnp.float32
    inv_std = lax.rsqrt(jnp.maximum(time_var, 1e-7).astype(f32))
    b0 = boundaries[0].astype(f32)
    brange = boundaries[NBOUND - 1].astype(f32) - b0
    istep = jnp.where(brange > 0, (NBOUND - 1) / brange, 0.0).astype(f32)
    params = jnp.stack([time_mean.astype(f32), inv_std, b0, istep])
    params16 = jnp.broadcast_to(params[:, None], (4, L))
    ttab_pad = jnp.pad(time_table, ((0, 0), (1, TW - 1 - D)))
    out = _sc_lookup(CUSTOMER_CODE, ACTION_ID, WEIGHT_int, TIMES,
                     customer_table, action_table, weight_table, ttab_pad,
                     boundaries, params16)
    return out[:, :OUT_D]


# speedup vs baseline: 3.8367x; 1.6371x over previous
import functools

import jax
import jax.numpy as jnp
from jax import lax
from jax.experimental import pallas as pl
from jax.experimental.pallas import tpu as pltpu
from jax.experimental.pallas import tpu_sc as plsc

B = 16384
D = 64
NCUST = 1000000
NBOUND = 1100
OUT_D = 3 * D + 1 + D
TW = 72
RW = 200
L = 16
NC = 2
NS = 16
NW = NC * NS
ROWS_PER_W = B // NW
NB = 128
NCHUNK = ROWS_PER_W // NB
TILE_C = 128
NTILE = NCUST // TILE_C
TAIL0 = NTILE * TILE_C

_mesh = plsc.VectorSubcoreMesh(core_axis_name="c", subcore_axis_name="s")


@functools.partial(
    pl.kernel,
    out_type=jax.ShapeDtypeStruct((B, D), jnp.float32),
    mesh=_mesh,
    compiler_params=pltpu.CompilerParams(use_tc_tiling_on_sc=True,
                                         needs_layout_passes=False),
    scratch_types=[
        pltpu.VMEM((ROWS_PER_W,), jnp.int32),
        pltpu.VMEM((D, TILE_C), jnp.float32),
        pltpu.VMEM((D, TILE_C), jnp.float32),
        pltpu.VMEM((D, D), jnp.float32),
        pltpu.VMEM((ROWS_PER_W, D), jnp.float32),
        pltpu.SemaphoreType.DMA,
        pltpu.SemaphoreType.DMA,
    ],
)
def _sc_customer(cidx_hbm, ctabT_hbm, tail_hbm, out_hbm,
                 cidx_s, tb0, tb1, tail_v, rows_v, sem0, sem1):
    wid = lax.axis_index("s") * NC + lax.axis_index("c")
    base_w = wid * ROWS_PER_W

    pltpu.sync_copy(cidx_hbm.at[pl.ds(base_w, ROWS_PER_W)], cidx_s)
    pltpu.sync_copy(tail_hbm, tail_v)

    def group(g, carry):
        gb = g * L
        cv = cidx_s[pl.ds(gb, L)]
        tcv = jnp.minimum(lax.shift_right_logical(cv, 7), NTILE - 1)

        def fire(l, buf, sem):
            off = pl.multiple_of(tcv[l] * TILE_C, TILE_C)
            pltpu.async_copy(ctabT_hbm.at[:, pl.ds(off, TILE_C)], buf, sem)

        def extract(l, buf, sem):
            pltpu.make_async_copy(ctabT_hbm.at[:, pl.ds(0, TILE_C)], buf,
                                  sem).wait()
            c = cv[l]
            use_tail = jnp.full((L,), c >= TAIL0, jnp.bool_)
            col_a = jnp.full((L,), jnp.minimum(c - tcv[l] * TILE_C,
                                               TILE_C - 1), jnp.int32)
            col_t = jnp.full((L,), jnp.clip(c - TAIL0, 0, D - 1), jnp.int32)
            rvec = gb + l + jnp.zeros((L,), jnp.int32)
            for k in range(D // L):
                fvec = k * L + lax.iota(jnp.int32, L)
                va = plsc.load_gather(buf, [fvec, col_a])
                vt = plsc.load_gather(tail_v, [col_t, fvec])
                plsc.store_scatter(rows_v, [rvec, fvec],
                                   jnp.where(use_tail, vt, va))

        fire(0, tb0, sem0)
        fire(1, tb1, sem1)
        for l in range(L - 2):
            buf, sem = (tb0, sem0) if l % 2 == 0 else (tb1, sem1)
            extract(l, buf, sem)
            fire(l + 2, buf, sem)
        extract(L - 2, tb0, sem0)
        extract(L - 1, tb1, sem1)
        return carry

    lax.fori_loop(0, ROWS_PER_W // L, group, 0)
    pltpu.sync_copy(rows_v, out_hbm.at[pl.ds(base_w, ROWS_PER_W)])


@functools.partial(
    pl.kernel,
    out_type=jax.ShapeDtypeStruct((B, RW), jnp.float32),
    mesh=_mesh,
    compiler_params=pltpu.CompilerParams(use_tc_tiling_on_sc=False,
                                         needs_layout_passes=False),
    scratch_types=[
        pltpu.VMEM((NB,), jnp.int32),
        pltpu.VMEM((NB,), jnp.int32),
        pltpu.VMEM((NB,), jnp.float32),
        pltpu.VMEM((NBOUND,), jnp.float32),
        pltpu.VMEM((4, L), jnp.float32),
        pltpu.VMEM((NB, D), jnp.float32),
        pltpu.VMEM((NB, D), jnp.float32),
        pltpu.VMEM((NB, TW), jnp.float32),
        pltpu.SemaphoreType.DMA,
    ],
)
def _sc_rest(aidx_hbm, widx_hbm, times_hbm,
             atab_hbm, wtab_hbm, ttab_hbm,
             bounds_hbm, params_hbm, out_hbm,
             aidx_v, widx_v, times_v, bounds_v, params_v,
             arows_v, wrows_v, trows_v, sem):
    wid = lax.axis_index("s") * NC + lax.axis_index("c")
    base_w = wid * ROWS_PER_W

    pltpu.sync_copy(bounds_hbm, bounds_v)
    pltpu.sync_copy(params_hbm, params_v)
    mean = params_v[0]
    inv = params_v[1]
    b0 = params_v[2]
    istep = params_v[3]

    def chunk_body(c, carry):
        r0 = base_w + c * NB
        pltpu.sync_copy(aidx_hbm.at[pl.ds(r0, NB)], aidx_v)
        pltpu.sync_copy(widx_hbm.at[pl.ds(r0, NB)], widx_v)
        pltpu.sync_copy(times_hbm.at[pl.ds(r0, NB)], times_v)

        g2 = pltpu.async_copy(atab_hbm.at[aidx_v], arows_v, sem)
        g3 = pltpu.async_copy(wtab_hbm.at[widx_v], wrows_v, sem)

        descs = []
        for i in range(NB // L):
            t = times_v[pl.ds(i * L, L)]
            g = ((t - b0) * istep + 0.5).astype(jnp.int32)
            m = jnp.clip(g - 2, 0, NBOUND - 4)
            cnt = jnp.zeros((L,), jnp.int32)
            for k in range(4):
                bk = plsc.load_gather(bounds_v, [m + k])
                cnt = cnt + jnp.where(bk < t, 1, 0)
            bins = m + cnt
            descs.append(pltpu.async_copy(
                ttab_hbm.at[bins], trows_v.at[pl.ds(i * L, L)], sem))

        g2.wait()
        g3.wait()
        for g in descs:
            g.wait()

        def norm(i, acc):
            t = times_v[pl.ds(i * L, L)]
            tn = (t - mean) * inv
            rows = i * L + lax.iota(jnp.int32, L)
            plsc.store_scatter(trows_v, [rows, jnp.zeros((L,), jnp.int32)], tn)
            return acc

        lax.fori_loop(0, NB // L, norm, 0)

        pltpu.sync_copy(arows_v, out_hbm.at[pl.ds(r0, NB), pl.ds(0, D)])
        pltpu.sync_copy(wrows_v, out_hbm.at[pl.ds(r0, NB), pl.ds(D, D)])
        pltpu.sync_copy(trows_v, out_hbm.at[pl.ds(r0, NB), pl.ds(2 * D, TW)])
        return carry

    lax.fori_loop(0, NCHUNK, chunk_body, 0)


def kernel(CUSTOMER_CODE, ACTION_ID, WEIGHT_int, TIMES, customer_table,
           action_table, weight_table, time_table, time_mean, time_var,
           boundaries):
    f32 = jnp.float32
    inv_std = lax.rsqrt(jnp.maximum(time_var, 1e-7).astype(f32))
    b0 = boundaries[0].astype(f32)
    brange = boundaries[NBOUND - 1].astype(f32) - b0
    istep = jnp.where(brange > 0, (NBOUND - 1) / brange, 0.0).astype(f32)
    params = jnp.stack([time_mean.astype(f32), inv_std, b0, istep])
    params16 = jnp.broadcast_to(params[:, None], (4, L))
    ttab_pad = jnp.pad(time_table, ((0, 0), (1, TW - 1 - D)))
    ctabT = customer_table.T
    tail = customer_table[TAIL0:]
    cust = _sc_customer(CUSTOMER_CODE, ctabT, tail)
    rest = _sc_rest(ACTION_ID, WEIGHT_int, TIMES, action_table, weight_table,
                    ttab_pad, boundaries, params16)
    return jnp.concatenate([cust, rest[:, :OUT_D - D]], axis=1)


# speedup vs baseline: 4.6523x; 1.2126x over previous
import functools

import jax
import jax.numpy as jnp
from jax import lax
from jax.experimental import pallas as pl
from jax.experimental.pallas import tpu as pltpu
from jax.experimental.pallas import tpu_sc as plsc

B = 16384
D = 64
NCUST = 1000000
NBOUND = 1100
OUT_D = 3 * D + 1 + D
TW = 72
RW = 200
L = 16
NC = 2
NS = 16
NW = NC * NS
ROWS_PER_W = B // NW
NB = 128
NCHUNK = ROWS_PER_W // NB
TILE_C = 128
NTILE = NCUST // TILE_C
TAIL0 = NTILE * TILE_C

_mesh = plsc.VectorSubcoreMesh(core_axis_name="c", subcore_axis_name="s")


@functools.partial(
    pl.kernel,
    out_type=jax.ShapeDtypeStruct((B, D), jnp.float32),
    mesh=_mesh,
    compiler_params=pltpu.CompilerParams(use_tc_tiling_on_sc=True,
                                         needs_layout_passes=False),
    scratch_types=[
        pltpu.VMEM((ROWS_PER_W,), jnp.int32),
        pltpu.VMEM((D, TILE_C), jnp.float32),
        pltpu.VMEM((D, TILE_C), jnp.float32),
        pltpu.VMEM((D, TILE_C), jnp.float32),
        pltpu.VMEM((D, TILE_C), jnp.float32),
        pltpu.VMEM((D, D), jnp.float32),
        pltpu.VMEM((ROWS_PER_W, D), jnp.float32),
        pltpu.SemaphoreType.DMA,
        pltpu.SemaphoreType.DMA,
        pltpu.SemaphoreType.DMA,
        pltpu.SemaphoreType.DMA,
    ],
)
def _sc_customer(cidx_hbm, ctabT_hbm, tail_hbm, out_hbm,
                 cidx_s, tb0, tb1, tb2, tb3, tail_v, rows_v,
                 sem0, sem1, sem2, sem3):
    wid = lax.axis_index("s") * NC + lax.axis_index("c")
    base_w = wid * ROWS_PER_W

    pltpu.sync_copy(cidx_hbm.at[pl.ds(base_w, ROWS_PER_W)], cidx_s)
    pltpu.sync_copy(tail_hbm, tail_v)

    def group(g, carry):
        gb = g * L
        cv = cidx_s[pl.ds(gb, L)]
        tcv = jnp.minimum(lax.shift_right_logical(cv, 7), NTILE - 1)

        def fire(l, buf, sem):
            off = pl.multiple_of(tcv[l] * TILE_C, TILE_C)
            pltpu.async_copy(ctabT_hbm.at[:, pl.ds(off, TILE_C)], buf, sem)

        def extract(l, buf, sem):
            pltpu.make_async_copy(ctabT_hbm.at[:, pl.ds(0, TILE_C)], buf,
                                  sem).wait()
            c = cv[l]
            use_tail = jnp.full((L,), c >= TAIL0, jnp.bool_)
            col_a = jnp.full((L,), jnp.minimum(c - tcv[l] * TILE_C,
                                               TILE_C - 1), jnp.int32)
            col_t = jnp.full((L,), jnp.clip(c - TAIL0, 0, D - 1), jnp.int32)
            rvec = gb + l + jnp.zeros((L,), jnp.int32)
            for k in range(D // L):
                fvec = k * L + lax.iota(jnp.int32, L)
                va = plsc.load_gather(buf, [fvec, col_a])
                vt = plsc.load_gather(tail_v, [col_t, fvec])
                plsc.store_scatter(rows_v, [rvec, fvec],
                                   jnp.where(use_tail, vt, va))

        bufs = [(tb0, sem0), (tb1, sem1), (tb2, sem2), (tb3, sem3)]
        for l in range(4):
            fire(l, *bufs[l])
        for l in range(L - 4):
            buf, sem = bufs[l % 4]
            extract(l, buf, sem)
            fire(l + 4, buf, sem)
        for l in range(L - 4, L):
            extract(l, *bufs[l % 4])
        return carry

    lax.fori_loop(0, ROWS_PER_W // L, group, 0)
    pltpu.sync_copy(rows_v, out_hbm.at[pl.ds(base_w, ROWS_PER_W)])


@functools.partial(
    pl.kernel,
    out_type=jax.ShapeDtypeStruct((B, RW), jnp.float32),
    mesh=_mesh,
    compiler_params=pltpu.CompilerParams(use_tc_tiling_on_sc=False,
                                         needs_layout_passes=False),
    scratch_types=[
        pltpu.VMEM((NB,), jnp.int32),
        pltpu.VMEM((NB,), jnp.int32),
        pltpu.VMEM((NB,), jnp.float32),
        pltpu.VMEM((NBOUND,), jnp.float32),
        pltpu.VMEM((4, L), jnp.float32),
        pltpu.VMEM((NB, D), jnp.float32),
        pltpu.VMEM((NB, D), jnp.float32),
        pltpu.VMEM((NB, TW), jnp.float32),
        pltpu.SemaphoreType.DMA,
    ],
)
def _sc_rest(aidx_hbm, widx_hbm, times_hbm,
             atab_hbm, wtab_hbm, ttab_hbm,
             bounds_hbm, params_hbm, out_hbm,
             aidx_v, widx_v, times_v, bounds_v, params_v,
             arows_v, wrows_v, trows_v, sem):
    wid = lax.axis_index("s") * NC + lax.axis_index("c")
    base_w = wid * ROWS_PER_W

    pltpu.sync_copy(bounds_hbm, bounds_v)
    pltpu.sync_copy(params_hbm, params_v)
    mean = params_v[0]
    inv = params_v[1]
    b0 = params_v[2]
    istep = params_v[3]

    def chunk_body(c, carry):
        r0 = base_w + c * NB
        pltpu.sync_copy(aidx_hbm.at[pl.ds(r0, NB)], aidx_v)
        pltpu.sync_copy(widx_hbm.at[pl.ds(r0, NB)], widx_v)
        pltpu.sync_copy(times_hbm.at[pl.ds(r0, NB)], times_v)

        g2 = pltpu.async_copy(atab_hbm.at[aidx_v], arows_v, sem)
        g3 = pltpu.async_copy(wtab_hbm.at[widx_v], wrows_v, sem)

        descs = []
        for i in range(NB // L):
            t = times_v[pl.ds(i * L, L)]
            g = ((t - b0) * istep + 0.5).astype(jnp.int32)
            m = jnp.clip(g - 2, 0, NBOUND - 4)
            cnt = jnp.zeros((L,), jnp.int32)
            for k in range(4):
                bk = plsc.load_gather(bounds_v, [m + k])
                cnt = cnt + jnp.where(bk < t, 1, 0)
            bins = m + cnt
            descs.append(pltpu.async_copy(
                ttab_hbm.at[bins], trows_v.at[pl.ds(i * L, L)], sem))

        g2.wait()
        g3.wait()
        for g in descs:
            g.wait()

        def norm(i, acc):
            t = times_v[pl.ds(i * L, L)]
            tn = (t - mean) * inv
            rows = i * L + lax.iota(jnp.int32, L)
            plsc.store_scatter(trows_v, [rows, jnp.zeros((L,), jnp.int32)], tn)
            return acc

        lax.fori_loop(0, NB // L, norm, 0)

        pltpu.sync_copy(arows_v, out_hbm.at[pl.ds(r0, NB), pl.ds(0, D)])
        pltpu.sync_copy(wrows_v, out_hbm.at[pl.ds(r0, NB), pl.ds(D, D)])
        pltpu.sync_copy(trows_v, out_hbm.at[pl.ds(r0, NB), pl.ds(2 * D, TW)])
        return carry

    lax.fori_loop(0, NCHUNK, chunk_body, 0)


def kernel(CUSTOMER_CODE, ACTION_ID, WEIGHT_int, TIMES, customer_table,
           action_table, weight_table, time_table, time_mean, time_var,
           boundaries):
    f32 = jnp.float32
    inv_std = lax.rsqrt(jnp.maximum(time_var, 1e-7).astype(f32))
    b0 = boundaries[0].astype(f32)
    brange = boundaries[NBOUND - 1].astype(f32) - b0
    istep = jnp.where(brange > 0, (NBOUND - 1) / brange, 0.0).astype(f32)
    params = jnp.stack([time_mean.astype(f32), inv_std, b0, istep])
    params16 = jnp.broadcast_to(params[:, None], (4, L))
    ttab_pad = jnp.pad(time_table, ((0, 0), (1, TW - 1 - D)))
    ctabT = customer_table.T
    tail = customer_table[TAIL0:]
    cust = _sc_customer(CUSTOMER_CODE, ctabT, tail)
    rest = _sc_rest(ACTION_ID, WEIGHT_int, TIMES, action_table, weight_table,
                    ttab_pad, boundaries, params16)
    return jnp.concatenate([cust, rest[:, :OUT_D - D]], axis=1)


# speedup vs baseline: 5.3005x; 1.1393x over previous
import functools

import jax
import jax.numpy as jnp
from jax import lax
from jax.experimental import pallas as pl
from jax.experimental.pallas import tpu as pltpu
from jax.experimental.pallas import tpu_sc as plsc

B = 16384
D = 64
NCUST = 1000000
NBOUND = 1100
OUT_D = 3 * D + 1 + D
TW = 72
RW = 200
L = 16
NC = 2
NS = 16
NW = NC * NS
ROWS_PER_W = B // NW
NB = 128
NCHUNK = ROWS_PER_W // NB
TILE_C = 128
NTILE = NCUST // TILE_C
TAIL0 = NTILE * TILE_C

_mesh = plsc.VectorSubcoreMesh(core_axis_name="c", subcore_axis_name="s")


@functools.partial(
    pl.kernel,
    out_type=jax.ShapeDtypeStruct((B, D), jnp.float32),
    mesh=_mesh,
    compiler_params=pltpu.CompilerParams(use_tc_tiling_on_sc=True,
                                         needs_layout_passes=False),
    scratch_types=[
        pltpu.VMEM((ROWS_PER_W,), jnp.int32),
        pltpu.VMEM((D, TILE_C), jnp.float32),
        pltpu.VMEM((D, TILE_C), jnp.float32),
        pltpu.VMEM((D, TILE_C), jnp.float32),
        pltpu.VMEM((D, TILE_C), jnp.float32),
        pltpu.VMEM((D, D), jnp.float32),
        pltpu.VMEM((ROWS_PER_W, D), jnp.float32),
        pltpu.SemaphoreType.DMA,
        pltpu.SemaphoreType.DMA,
        pltpu.SemaphoreType.DMA,
        pltpu.SemaphoreType.DMA,
    ],
)
def _sc_customer(cidx_hbm, ctabT_hbm, tail_hbm, out_hbm,
                 cidx_s, tb0, tb1, tb2, tb3, tail_v, rows_v,
                 sem0, sem1, sem2, sem3):
    wid = lax.axis_index("s") * NC + lax.axis_index("c")
    base_w = wid * ROWS_PER_W

    pltpu.sync_copy(cidx_hbm.at[pl.ds(base_w, ROWS_PER_W)], cidx_s)
    pltpu.sync_copy(tail_hbm, tail_v)

    def group(g, carry):
        gb = g * L
        cv = cidx_s[pl.ds(gb, L)]
        tcv = jnp.minimum(lax.shift_right_logical(cv, 7), NTILE - 1)

        def fire(l, buf, sem):
            off = pl.multiple_of(tcv[l] * TILE_C, TILE_C)
            pltpu.async_copy(ctabT_hbm.at[:, pl.ds(off, TILE_C)], buf, sem)

        def extract(l, buf, sem):
            pltpu.make_async_copy(ctabT_hbm.at[:, pl.ds(0, TILE_C)], buf,
                                  sem).wait()
            c = cv[l]
            use_tail = jnp.full((L,), c >= TAIL0, jnp.bool_)
            col_a = jnp.full((L,), jnp.minimum(c - tcv[l] * TILE_C,
                                               TILE_C - 1), jnp.int32)
            col_t = jnp.full((L,), jnp.clip(c - TAIL0, 0, D - 1), jnp.int32)
            rvec = gb + l + jnp.zeros((L,), jnp.int32)
            for k in range(D // L):
                fvec = k * L + lax.iota(jnp.int32, L)
                va = plsc.load_gather(buf, [fvec, col_a])
                vt = plsc.load_gather(tail_v, [col_t, fvec])
                plsc.store_scatter(rows_v, [rvec, fvec],
                                   jnp.where(use_tail, vt, va))

        bufs = [(tb0, sem0), (tb1, sem1), (tb2, sem2), (tb3, sem3)]
        for l in range(4):
            fire(l, *bufs[l])
        for l in range(L - 4):
            buf, sem = bufs[l % 4]
            extract(l, buf, sem)
            fire(l + 4, buf, sem)
        for l in range(L - 4, L):
            extract(l, *bufs[l % 4])
        return carry

    lax.fori_loop(0, ROWS_PER_W // L, group, 0)
    pltpu.sync_copy(rows_v, out_hbm.at[pl.ds(base_w, ROWS_PER_W)])


TCB = 256
NBLK = B // TCB
NACT = 1000
NWT = 100


def _tc_rest_body(aidx_ref, widx_ref, times_ref, atab_ref, wtab_ref,
                  ttab_ref, bounds_ref, params_ref, out_ref):
    f32 = jnp.float32
    aidx = aidx_ref[0, 0, :]
    widx = widx_ref[0, 0, :]
    t = times_ref[0, 0, :]
    bounds = bounds_ref[0, :]
    mean = params_ref[0, 0]
    inv = params_ref[0, 1]

    dn = (((1,), (0,)), ((), ()))
    oh_a = (aidx[:, None] ==
            lax.broadcasted_iota(jnp.int32, (1, NACT), 1)).astype(f32)
    act = lax.dot_general(oh_a, atab_ref[...], dn,
                          precision=lax.Precision.HIGHEST,
                          preferred_element_type=f32)
    oh_w = (widx[:, None] ==
            lax.broadcasted_iota(jnp.int32, (1, NWT), 1)).astype(f32)
    wt = lax.dot_general(oh_w, wtab_ref[...], dn,
                         precision=lax.Precision.HIGHEST,
                         preferred_element_type=f32)
    bins = jnp.sum((bounds[None, :] < t[:, None]).astype(jnp.int32), axis=1)
    oh_t = (bins[:, None] ==
            lax.broadcasted_iota(jnp.int32, (1, NBOUND + 1), 1)).astype(f32)
    temb = lax.dot_general(oh_t, ttab_ref[...], dn,
                           precision=lax.Precision.HIGHEST,
                           preferred_element_type=f32)
    tn = (t - mean) * inv
    out_ref[:, 0:D] = act
    out_ref[:, D:2 * D] = wt
    out_ref[:, 2 * D:2 * D + 1] = tn[:, None]
    out_ref[:, 2 * D + 1:] = temb


_tc_rest = pl.pallas_call(
    _tc_rest_body,
    grid=(NBLK,),
    in_specs=[
        pl.BlockSpec((1, 1, TCB), lambda i: (i, 0, 0)),
        pl.BlockSpec((1, 1, TCB), lambda i: (i, 0, 0)),
        pl.BlockSpec((1, 1, TCB), lambda i: (i, 0, 0)),
        pl.BlockSpec((NACT, D), lambda i: (0, 0)),
        pl.BlockSpec((NWT, D), lambda i: (0, 0)),
        pl.BlockSpec((NBOUND + 1, D), lambda i: (0, 0)),
        pl.BlockSpec((1, NBOUND), lambda i: (0, 0)),
        pl.BlockSpec((1, 2), lambda i: (0, 0)),
    ],
    out_specs=pl.BlockSpec((TCB, OUT_D - D), lambda i: (i, 0)),
    out_shape=jax.ShapeDtypeStruct((B, OUT_D - D), jnp.float32),
)


def kernel(CUSTOMER_CODE, ACTION_ID, WEIGHT_int, TIMES, customer_table,
           action_table, weight_table, time_table, time_mean, time_var,
           boundaries):
    f32 = jnp.float32
    inv_std = lax.rsqrt(jnp.maximum(time_var, 1e-7).astype(f32))
    params = jnp.stack([time_mean.astype(f32), inv_std])[None, :]
    ctabT = customer_table.T
    tail = customer_table[TAIL0:]
    cust = _sc_customer(CUSTOMER_CODE, ctabT, tail)
    rest = _tc_rest(ACTION_ID.reshape(NBLK, 1, TCB),
                    WEIGHT_int.reshape(NBLK, 1, TCB),
                    TIMES.reshape(NBLK, 1, TCB),
                    action_table, weight_table, time_table,
                    boundaries[None, :], params)
    return jnp.concatenate([cust, rest], axis=1)


# speedup vs baseline: 5.8937x; 1.1119x over previous
import functools

import jax
import jax.numpy as jnp
from jax import lax
from jax.experimental import pallas as pl
from jax.experimental.pallas import tpu as pltpu
from jax.experimental.pallas import tpu_sc as plsc

B = 16384
D = 64
NCUST = 1000000
NBOUND = 1100
OUT_D = 3 * D + 1 + D
TW = 72
RW = 200
L = 16
NC = 2
NS = 16
NW = NC * NS
ROWS_PER_W = B // NW
NB = 128
NCHUNK = ROWS_PER_W // NB
TILE_C = 128
NTILE = NCUST // TILE_C
TAIL0 = NTILE * TILE_C

_mesh = plsc.VectorSubcoreMesh(core_axis_name="c", subcore_axis_name="s")


@functools.partial(
    pl.kernel,
    out_type=jax.ShapeDtypeStruct((B, D), jnp.float32),
    mesh=_mesh,
    compiler_params=pltpu.CompilerParams(use_tc_tiling_on_sc=True,
                                         needs_layout_passes=False),
    scratch_types=[
        pltpu.VMEM((ROWS_PER_W,), jnp.int32),
        pltpu.VMEM((D, TILE_C), jnp.float32),
        pltpu.VMEM((D, TILE_C), jnp.float32),
        pltpu.VMEM((D, TILE_C), jnp.float32),
        pltpu.VMEM((D, TILE_C), jnp.float32),
        pltpu.VMEM((D, TILE_C), jnp.float32),
        pltpu.VMEM((D, TILE_C), jnp.float32),
        pltpu.VMEM((D, TILE_C), jnp.float32),
        pltpu.VMEM((D, TILE_C), jnp.float32),
        pltpu.VMEM((D, D), jnp.float32),
        pltpu.VMEM((ROWS_PER_W // 2, D), jnp.float32),
        pltpu.SemaphoreType.DMA,
        pltpu.SemaphoreType.DMA,
        pltpu.SemaphoreType.DMA,
        pltpu.SemaphoreType.DMA,
        pltpu.SemaphoreType.DMA,
        pltpu.SemaphoreType.DMA,
        pltpu.SemaphoreType.DMA,
        pltpu.SemaphoreType.DMA,
    ],
)
def _sc_customer(cidx_hbm, ctabT_hbm, tail_hbm, out_hbm,
                 cidx_s, tb0, tb1, tb2, tb3, tb4, tb5, tb6, tb7,
                 tail_v, rows_v,
                 sem0, sem1, sem2, sem3, sem4, sem5, sem6, sem7):
    wid = lax.axis_index("s") * NC + lax.axis_index("c")
    base_w = wid * ROWS_PER_W

    pltpu.sync_copy(cidx_hbm.at[pl.ds(base_w, ROWS_PER_W)], cidx_s)
    pltpu.sync_copy(tail_hbm, tail_v)

    def group(g, carry):
        gb = g * L
        hb = (g % (ROWS_PER_W // L // 2)) * L
        cv = cidx_s[pl.ds(gb, L)]
        tcv = jnp.minimum(lax.shift_right_logical(cv, 7), NTILE - 1)

        def fire(l, buf, sem):
            off = pl.multiple_of(tcv[l] * TILE_C, TILE_C)
            pltpu.async_copy(ctabT_hbm.at[:, pl.ds(off, TILE_C)], buf, sem)

        def extract(l, buf, sem):
            pltpu.make_async_copy(ctabT_hbm.at[:, pl.ds(0, TILE_C)], buf,
                                  sem).wait()
            c = cv[l]
            use_tail = jnp.full((L,), c >= TAIL0, jnp.bool_)
            col_a = jnp.full((L,), jnp.minimum(c - tcv[l] * TILE_C,
                                               TILE_C - 1), jnp.int32)
            col_t = jnp.full((L,), jnp.clip(c - TAIL0, 0, D - 1), jnp.int32)
            rvec = hb + l + jnp.zeros((L,), jnp.int32)
            for k in range(D // L):
                fvec = k * L + lax.iota(jnp.int32, L)
                va = plsc.load_gather(buf, [fvec, col_a])
                vt = plsc.load_gather(tail_v, [col_t, fvec])
                plsc.store_scatter(rows_v, [rvec, fvec],
                                   jnp.where(use_tail, vt, va))

        bufs = [(tb0, sem0), (tb1, sem1), (tb2, sem2), (tb3, sem3),
                (tb4, sem4), (tb5, sem5), (tb6, sem6), (tb7, sem7)]
        for l in range(8):
            fire(l, *bufs[l])
        for l in range(L - 8):
            buf, sem = bufs[l % 8]
            extract(l, buf, sem)
            fire(l + 8, buf, sem)
        for l in range(L - 8, L):
            extract(l, *bufs[l % 8])
        return carry

    HGRP = ROWS_PER_W // L // 2

    def half(h, carry):
        lax.fori_loop(h * HGRP, (h + 1) * HGRP, group, 0)
        pltpu.sync_copy(
            rows_v,
            out_hbm.at[pl.ds(base_w + h * (ROWS_PER_W // 2),
                             ROWS_PER_W // 2)])
        return carry

    lax.fori_loop(0, 2, half, 0)


TCB = 256
NBLK = B // TCB
NACT = 1000
NWT = 100


def _tc_rest_body(aidx_ref, widx_ref, times_ref, atab_ref, wtab_ref,
                  ttab_ref, bounds_ref, params_ref, out_ref):
    f32 = jnp.float32
    aidx = aidx_ref[0, 0, :]
    widx = widx_ref[0, 0, :]
    t = times_ref[0, 0, :]
    bounds = bounds_ref[0, :]
    mean = params_ref[0, 0]
    inv = params_ref[0, 1]

    dn = (((1,), (0,)), ((), ()))
    oh_a = (aidx[:, None] ==
            lax.broadcasted_iota(jnp.int32, (1, NACT), 1)).astype(f32)
    act = lax.dot_general(oh_a, atab_ref[...], dn,
                          precision=lax.Precision.HIGHEST,
                          preferred_element_type=f32)
    oh_w = (widx[:, None] ==
            lax.broadcasted_iota(jnp.int32, (1, NWT), 1)).astype(f32)
    wt = lax.dot_general(oh_w, wtab_ref[...], dn,
                         precision=lax.Precision.HIGHEST,
                         preferred_element_type=f32)
    bins = jnp.sum((bounds[None, :] < t[:, None]).astype(jnp.int32), axis=1)
    oh_t = (bins[:, None] ==
            lax.broadcasted_iota(jnp.int32, (1, NBOUND + 1), 1)).astype(f32)
    temb = lax.dot_general(oh_t, ttab_ref[...], dn,
                           precision=lax.Precision.HIGHEST,
                           preferred_element_type=f32)
    tn = (t - mean) * inv
    out_ref[:, 0:D] = act
    out_ref[:, D:2 * D] = wt
    out_ref[:, 2 * D:2 * D + 1] = tn[:, None]
    out_ref[:, 2 * D + 1:] = temb


_tc_rest = pl.pallas_call(
    _tc_rest_body,
    grid=(NBLK,),
    in_specs=[
        pl.BlockSpec((1, 1, TCB), lambda i: (i, 0, 0)),
        pl.BlockSpec((1, 1, TCB), lambda i: (i, 0, 0)),
        pl.BlockSpec((1, 1, TCB), lambda i: (i, 0, 0)),
        pl.BlockSpec((NACT, D), lambda i: (0, 0)),
        pl.BlockSpec((NWT, D), lambda i: (0, 0)),
        pl.BlockSpec((NBOUND + 1, D), lambda i: (0, 0)),
        pl.BlockSpec((1, NBOUND), lambda i: (0, 0)),
        pl.BlockSpec((1, 2), lambda i: (0, 0)),
    ],
    out_specs=pl.BlockSpec((TCB, OUT_D - D), lambda i: (i, 0)),
    out_shape=jax.ShapeDtypeStruct((B, OUT_D - D), jnp.float32),
)


def kernel(CUSTOMER_CODE, ACTION_ID, WEIGHT_int, TIMES, customer_table,
           action_table, weight_table, time_table, time_mean, time_var,
           boundaries):
    f32 = jnp.float32
    inv_std = lax.rsqrt(jnp.maximum(time_var, 1e-7).astype(f32))
    params = jnp.stack([time_mean.astype(f32), inv_std])[None, :]
    ctabT = customer_table.T
    tail = customer_table[TAIL0:]
    cust = _sc_customer(CUSTOMER_CODE, ctabT, tail)
    rest = _tc_rest(ACTION_ID.reshape(NBLK, 1, TCB),
                    WEIGHT_int.reshape(NBLK, 1, TCB),
                    TIMES.reshape(NBLK, 1, TCB),
                    action_table, weight_table, time_table,
                    boundaries[None, :], params)
    return jnp.concatenate([cust, rest], axis=1)
